# Initial kernel scaffold; baseline (speedup 1.0000x reference)
import functools

import jax
import jax.numpy as jnp
from jax import lax
from jax.experimental import pallas as pl
from jax.experimental.pallas import tpu as pltpu
from jax.experimental.pallas import tpu_sc as plsc

N = 10000
E = 320000
D = 128

NC = 2
NS = 16
L = 16
NW = NC * NS

E_PER_W = E // NW
CHUNK = 80
NCHUNK = E_PER_W // CHUNK
ROWS_PER_TILE = N // NS


def _matmul_body(x_ref, w_ref, o_ref):
    o_ref[...] = jnp.dot(x_ref[...], w_ref[...],
                         preferred_element_type=jnp.float32)


def _project(inputs, w):
    return pl.pallas_call(
        _matmul_body,
        grid=(10,),
        in_specs=[
            pl.BlockSpec((N // 10, D), lambda i: (i, 0)),
            pl.BlockSpec((D, D), lambda i: (0, 0)),
        ],
        out_specs=pl.BlockSpec((N // 10, D), lambda i: (i, 0)),
        out_shape=jax.ShapeDtypeStruct((N, D), jnp.float32),
    )(inputs, w)


def _add_body(p_ref, o_ref):
    o_ref[...] = p_ref[0] + p_ref[1]


def _combine(partials):
    return pl.pallas_call(
        _add_body,
        grid=(10,),
        in_specs=[pl.BlockSpec((2, N // 10, D), lambda i: (0, i, 0))],
        out_specs=pl.BlockSpec((N // 10, D), lambda i: (i, 0)),
        out_shape=jax.ShapeDtypeStruct((N, D), jnp.float32),
    )(partials)


def _sc_edge_body(h_hbm, row_hbm, col_hbm, val_hbm, zero_hbm, out_hbm,
                  colv, rowv, valv, rows, acc, sem):
    cid = lax.axis_index("c")
    sid = lax.axis_index("s")
    wid = sid * NC + cid

    zbase = sid * ROWS_PER_TILE
    pltpu.sync_copy(zero_hbm.at[pl.ds(zbase, ROWS_PER_TILE)],
                    acc.at[pl.ds(zbase, ROWS_PER_TILE)])
    plsc.subcore_barrier()

    def step(i, carry):
        base = wid * E_PER_W + i * CHUNK
        pltpu.sync_copy(col_hbm.at[pl.ds(base, CHUNK)], colv)
        pltpu.sync_copy(row_hbm.at[pl.ds(base, CHUNK)], rowv)
        pltpu.sync_copy(val_hbm.at[pl.ds(base, CHUNK)], valv)
        pltpu.async_copy(h_hbm.at[colv], rows, sem).wait()
        for e in range(CHUNK):
            vb = plsc.load_gather(valv, [jnp.full((L,), e, jnp.int32)])
            for j in range(D // L):
                sl = pl.ds(j * L, L)
                rows[e, sl] = rows[e, sl] * vb
        pltpu.sync_copy(rows, acc.at[rowv], add=True)
        return carry

    lax.fori_loop(0, NCHUNK, step, 0)

    plsc.subcore_barrier()
    pltpu.sync_copy(acc.at[pl.ds(zbase, ROWS_PER_TILE)],
                    out_hbm.at[cid, pl.ds(zbase, ROWS_PER_TILE)])


_sc_edge = functools.partial(
    pl.kernel,
    _sc_edge_body,
    out_type=jax.ShapeDtypeStruct((NC, N, D), jnp.float32),
    mesh=plsc.VectorSubcoreMesh(core_axis_name="c", subcore_axis_name="s",
                                num_cores=NC, num_subcores=NS),
    scratch_types=[
        pltpu.VMEM((CHUNK,), jnp.int32),
        pltpu.VMEM((CHUNK,), jnp.int32),
        pltpu.VMEM((CHUNK,), jnp.float32),
        pltpu.VMEM((CHUNK, D), jnp.float32),
        pltpu.VMEM_SHARED((N, D), jnp.float32),
        pltpu.SemaphoreType.DMA,
    ],
)()


def kernel(inputs, edge_row, edge_col, edge_val, W):
    h = _project(inputs, W)
    zeros = jnp.zeros((N, D), jnp.float32)
    partials = _sc_edge(h, edge_row, edge_col, edge_val, zeros)
    return _combine(partials)


# speedup vs baseline: 4.1276x; 4.1276x over previous
import functools

import jax
import jax.numpy as jnp
from jax import lax
from jax.experimental import pallas as pl
from jax.experimental.pallas import tpu as pltpu
from jax.experimental.pallas import tpu_sc as plsc

N = 10000
E = 320000
D = 128

NC = 2
NS = 16
L = 16
NW = NC * NS

E_PER_W = E // NW
CHUNK = 80
NCHUNK = E_PER_W // CHUNK
NP = 10240
ROWS_PER_TILE = NP // NS


def _matmul_body(x_ref, w_ref, o_ref):
    o_ref[...] = jnp.dot(x_ref[...], w_ref[...],
                         preferred_element_type=jnp.float32)


def _project(inputs, w):
    return pl.pallas_call(
        _matmul_body,
        grid=(10,),
        in_specs=[
            pl.BlockSpec((N // 10, D), lambda i: (i, 0)),
            pl.BlockSpec((D, D), lambda i: (0, 0)),
        ],
        out_specs=pl.BlockSpec((N // 10, D), lambda i: (i, 0)),
        out_shape=jax.ShapeDtypeStruct((N, D), jnp.float32),
    )(inputs, w)


def _add_body(p_ref, o_ref):
    o_ref[...] = p_ref[0] + p_ref[1]


def _combine(partials):
    return pl.pallas_call(
        _add_body,
        grid=(10,),
        in_specs=[pl.BlockSpec((2, N // 10, D), lambda i: (0, i, 0))],

        out_specs=pl.BlockSpec((N // 10, D), lambda i: (i, 0)),
        out_shape=jax.ShapeDtypeStruct((N, D), jnp.float32),
    )(partials)


def _sc_edge_body(h_hbm, row_hbm, col_hbm, val_hbm, zero_hbm, out_hbm,
                  colv, rowv, valv, rows, acc, sem):
    cid = lax.axis_index("c")
    sid = lax.axis_index("s")
    wid = sid * NC + cid

    zbase = sid * ROWS_PER_TILE
    pltpu.sync_copy(zero_hbm.at[pl.ds(zbase, ROWS_PER_TILE)],
                    acc.at[pl.ds(zbase, ROWS_PER_TILE)])
    plsc.subcore_barrier()

    def step(i, carry):
        base = wid * E_PER_W + i * CHUNK
        pltpu.sync_copy(col_hbm.at[pl.ds(base, CHUNK)], colv)
        pltpu.sync_copy(row_hbm.at[pl.ds(base, CHUNK)], rowv)
        pltpu.sync_copy(val_hbm.at[pl.ds(base, CHUNK)], valv)
        pltpu.async_copy(h_hbm.at[colv], rows, sem).wait()
        for g in range(CHUNK // L):
            vv = valv[pl.ds(g * L, L)]
            for t in range(L):
                e = g * L + t
                vb = lax.gather(
                    vv, jnp.full((L, 1), t, jnp.int32),
                    lax.GatherDimensionNumbers(
                        offset_dims=(), collapsed_slice_dims=(0,),
                        start_index_map=(0,)),
                    (1,), mode=lax.GatherScatterMode.PROMISE_IN_BOUNDS)
                for j in range(D // L):
                    sl = pl.ds(j * L, L)
                    rows[e, sl] = rows[e, sl] * vb
        pltpu.sync_copy(rows, acc.at[rowv], add=True)
        return carry

    lax.fori_loop(0, NCHUNK, step, 0)

    plsc.subcore_barrier()
    pltpu.sync_copy(acc.at[pl.ds(zbase, ROWS_PER_TILE)],
                    out_hbm.at[cid, pl.ds(zbase, ROWS_PER_TILE)])


_sc_edge = pl.kernel(
    _sc_edge_body,
    out_type=jax.ShapeDtypeStruct((NC, NP, D), jnp.float32),
    mesh=plsc.VectorSubcoreMesh(core_axis_name="c", subcore_axis_name="s",
                                num_cores=NC, num_subcores=NS),
    scratch_types=[
        pltpu.VMEM((CHUNK,), jnp.int32),
        pltpu.VMEM((CHUNK,), jnp.int32),
        pltpu.VMEM((CHUNK,), jnp.float32),
        pltpu.VMEM((CHUNK, D), jnp.float32),
        pltpu.VMEM_SHARED((NP, D), jnp.float32),
        pltpu.SemaphoreType.DMA,
    ],
)


def kernel(inputs, edge_row, edge_col, edge_val, W):
    h = _project(inputs, W)
    zeros = jnp.zeros((NP, D), jnp.float32)
    partials = _sc_edge(h, edge_row, edge_col, edge_val, zeros)
    return _combine(partials)


# speedup vs baseline: 7.4155x; 1.7966x over previous
import jax
import jax.numpy as jnp
from jax import lax
from jax.experimental import pallas as pl
from jax.experimental.pallas import tpu as pltpu
from jax.experimental.pallas import tpu_sc as plsc

N = 10000
E = 320000
D = 128

NC = 2
NS = 16
L = 16
NW = NC * NS

E_PER_W = E // NW
CHUNK = 40
NCHUNK = E_PER_W // CHUNK
NBUF = 5
NP = 10240
ROWS_PER_TILE = NP // NS


def _matmul_body(x_ref, w_ref, o_ref):
    o_ref[...] = jnp.dot(x_ref[...], w_ref[...],
                         preferred_element_type=jnp.float32)


def _project(inputs, w):
    return pl.pallas_call(
        _matmul_body,
        grid=(10,),
        in_specs=[
            pl.BlockSpec((N // 10, D), lambda i: (i, 0)),
            pl.BlockSpec((D, D), lambda i: (0, 0)),
        ],
        out_specs=pl.BlockSpec((N // 10, D), lambda i: (i, 0)),
        out_shape=jax.ShapeDtypeStruct((N, D), jnp.float32),
    )(inputs, w)


def _add_body(p_ref, o_ref):
    o_ref[...] = p_ref[0] + p_ref[1]


def _combine(partials):
    return pl.pallas_call(
        _add_body,
        grid=(10,),
        in_specs=[pl.BlockSpec((2, N // 10, D), lambda i: (0, i, 0))],
        out_specs=pl.BlockSpec((N // 10, D), lambda i: (i, 0)),
        out_shape=jax.ShapeDtypeStruct((N, D), jnp.float32),
    )(partials)


def _bcast_lane(vv, t):
    return lax.gather(
        vv, jnp.full((L, 1), t, jnp.int32),
        lax.GatherDimensionNumbers(
            offset_dims=(), collapsed_slice_dims=(0,), start_index_map=(0,)),
        (1,), mode=lax.GatherScatterMode.PROMISE_IN_BOUNDS)


def _sc_edge_body(h_hbm, erow_hbm, ecol_hbm, eval_hbm, zero_hbm, out_hbm,
                  *refs):
    cr = refs[0:NBUF]
    rr = refs[NBUF:2 * NBUF]
    vr = refs[2 * NBUF:3 * NBUF]
    bufs = refs[3 * NBUF:4 * NBUF]
    acc = refs[4 * NBUF]
    si = refs[4 * NBUF + 1:4 * NBUF + 1 + NBUF]
    sg = refs[4 * NBUF + 1 + NBUF:4 * NBUF + 1 + 2 * NBUF]
    ss = refs[4 * NBUF + 1 + 2 * NBUF:4 * NBUF + 1 + 3 * NBUF]

    cid = lax.axis_index("c")
    sid = lax.axis_index("s")
    wid = sid * NC + cid

    zbase = sid * ROWS_PER_TILE
    pltpu.sync_copy(zero_hbm.at[pl.ds(zbase, ROWS_PER_TILE)],
                    acc.at[pl.ds(zbase, ROWS_PER_TILE)])
    plsc.subcore_barrier()

    def fire_idx(j, b):
        pltpu.async_copy(ecol_hbm.at[wid, j], cr[b], si[b])
        pltpu.async_copy(erow_hbm.at[wid, j], rr[b], si[b])
        pltpu.async_copy(eval_hbm.at[wid, j], vr[b], si[b])

    def wait_idx(j, b):
        pltpu.make_async_copy(ecol_hbm.at[wid, j], cr[b], si[b]).wait()
        pltpu.make_async_copy(erow_hbm.at[wid, j], rr[b], si[b]).wait()
        pltpu.make_async_copy(eval_hbm.at[wid, j], vr[b], si[b]).wait()

    def fire_gather(j, b):
        pltpu.async_copy(h_hbm.at[cr[b]], bufs[b], sg[b])

    def wait_gather(j, b):
        pltpu.make_async_copy(h_hbm.at[cr[b]], bufs[b], sg[b]).wait()

    def drain_scatter(j, b):
        pltpu.make_async_copy(bufs[b], acc.at[rr[b]], ss[b]).wait()

    fire_idx(0, 0)
    fire_idx(1, 1)
    fire_idx(2, 2)
    wait_idx(0, 0)
    fire_gather(0, 0)
    wait_idx(1, 1)
    fire_gather(1, 1)

    def process(i, b):
        buf = bufs[b]
        wait_gather(i, b)
        done = 0
        while done < CHUNK:
            off = min(done, CHUNK - L)
            vv = vr[b][pl.ds(off, L)]
            for t in range(done - off, L):
                e = off + t
                vb = _bcast_lane(vv, t)
                for j in range(D // L):
                    sl = pl.ds(j * L, L)
                    buf[e, sl] = buf[e, sl] * vb
            done = off + L
        pltpu.async_copy(buf, acc.at[rr[b]], ss[b], add=True)

        b3 = (b + 3) % NBUF

        @pl.when(i + 3 < NCHUNK)
        def _idx_prefetch():
            @pl.when(i >= 2)
            def _drain():
                drain_scatter(i - 2, b3)
            fire_idx(i + 3, b3)

        b2 = (b + 2) % NBUF

        @pl.when(i + 2 < NCHUNK)
        def _gather_prefetch():
            wait_idx(i + 2, b2)
            fire_gather(i + 2, b2)

    def outer(k, carry):
        for b in range(NBUF):
            process(k * NBUF + b, b)
        return carry

    lax.fori_loop(0, NCHUNK // NBUF, outer, 0)

    for b in range(NBUF):
        drain_scatter(NCHUNK - NBUF + b, b)

    plsc.subcore_barrier()
    pltpu.sync_copy(acc.at[pl.ds(zbase, ROWS_PER_TILE)],
                    out_hbm.at[cid, pl.ds(zbase, ROWS_PER_TILE)])


_sc_edge = pl.kernel(
    _sc_edge_body,
    out_type=jax.ShapeDtypeStruct((NC, NP, D), jnp.float32),
    mesh=plsc.VectorSubcoreMesh(core_axis_name="c", subcore_axis_name="s",
                                num_cores=NC, num_subcores=NS),
    scratch_types=(
        [pltpu.VMEM((CHUNK,), jnp.int32) for _ in range(NBUF)]
        + [pltpu.VMEM((CHUNK,), jnp.int32) for _ in range(NBUF)]
        + [pltpu.VMEM((CHUNK,), jnp.float32) for _ in range(NBUF)]
        + [pltpu.VMEM((CHUNK, D), jnp.float32) for _ in range(NBUF)]
        + [pltpu.VMEM_SHARED((NP, D), jnp.float32)]
        + [pltpu.SemaphoreType.DMA for _ in range(3 * NBUF)]
    ),
)


def kernel(inputs, edge_row, edge_col, edge_val, W):
    h = _project(inputs, W)
    zeros = jnp.zeros((NP, D), jnp.float32)
    erow3 = edge_row.reshape(NW, NCHUNK, CHUNK)
    ecol3 = edge_col.reshape(NW, NCHUNK, CHUNK)
    eval3 = edge_val.reshape(NW, NCHUNK, CHUNK)
    partials = _sc_edge(h, erow3, ecol3, eval3, zeros)
    return _combine(partials)


# speedup vs baseline: 8.3363x; 1.1242x over previous
import jax
import jax.numpy as jnp
from jax import lax
from jax.experimental import pallas as pl
from jax.experimental.pallas import tpu as pltpu
from jax.experimental.pallas import tpu_sc as plsc

N = 10000
E = 320000
D = 128

NC = 2
NS = 16
L = 16
NW = NC * NS

E_PER_W = E // NW
CHUNK = 80
NCHUNK = E_PER_W // CHUNK
NBUF = 4
NP = 10240
ROWS_PER_TILE = NP // NS


def _matmul_body(x_ref, w_ref, o_ref):
    o_ref[...] = jnp.dot(x_ref[...], w_ref[...],
                         preferred_element_type=jnp.float32)


def _project(inputs, w):
    return pl.pallas_call(
        _matmul_body,
        grid=(10,),
        in_specs=[
            pl.BlockSpec((N // 10, D), lambda i: (i, 0)),
            pl.BlockSpec((D, D), lambda i: (0, 0)),
        ],
        out_specs=pl.BlockSpec((N // 10, D), lambda i: (i, 0)),
        out_shape=jax.ShapeDtypeStruct((N, D), jnp.float32),
    )(inputs, w)


def _add_body(p_ref, o_ref):
    o_ref[...] = p_ref[0] + p_ref[1]


def _combine(partials):
    return pl.pallas_call(
        _add_body,
        grid=(10,),
        in_specs=[pl.BlockSpec((2, N // 10, D), lambda i: (0, i, 0))],
        out_specs=pl.BlockSpec((N // 10, D), lambda i: (i, 0)),
        out_shape=jax.ShapeDtypeStruct((N, D), jnp.float32),
    )(partials)


def _bcast_lane(vv, t):
    return lax.gather(
        vv, jnp.full((L, 1), t, jnp.int32),
        lax.GatherDimensionNumbers(
            offset_dims=(), collapsed_slice_dims=(0,), start_index_map=(0,)),
        (1,), mode=lax.GatherScatterMode.PROMISE_IN_BOUNDS)


def _sc_edge_body(h_hbm, erow_hbm, ecol_hbm, eval_hbm, zero_hbm, out_hbm,
                  *refs):
    cr = refs[0:NBUF]
    rr = refs[NBUF:2 * NBUF]
    vr = refs[2 * NBUF:3 * NBUF]
    bufs = refs[3 * NBUF:4 * NBUF]
    acc = refs[4 * NBUF]
    si = refs[4 * NBUF + 1:4 * NBUF + 1 + NBUF]
    sg = refs[4 * NBUF + 1 + NBUF:4 * NBUF + 1 + 2 * NBUF]
    ss = refs[4 * NBUF + 1 + 2 * NBUF:4 * NBUF + 1 + 3 * NBUF]

    cid = lax.axis_index("c")
    sid = lax.axis_index("s")
    wid = sid * NC + cid

    zbase = sid * ROWS_PER_TILE
    pltpu.sync_copy(zero_hbm.at[pl.ds(zbase, ROWS_PER_TILE)],
                    acc.at[pl.ds(zbase, ROWS_PER_TILE)])
    plsc.subcore_barrier()

    def fire_idx(j, b):
        pltpu.async_copy(ecol_hbm.at[wid, j], cr[b], si[b])
        pltpu.async_copy(erow_hbm.at[wid, j], rr[b], si[b])
        pltpu.async_copy(eval_hbm.at[wid, j], vr[b], si[b])

    def wait_idx(j, b):
        pltpu.make_async_copy(ecol_hbm.at[wid, j], cr[b], si[b]).wait()
        pltpu.make_async_copy(erow_hbm.at[wid, j], rr[b], si[b]).wait()
        pltpu.make_async_copy(eval_hbm.at[wid, j], vr[b], si[b]).wait()

    def fire_gather(j, b):
        pltpu.async_copy(h_hbm.at[cr[b]], bufs[b], sg[b])

    def wait_gather(j, b):
        pltpu.make_async_copy(h_hbm.at[cr[b]], bufs[b], sg[b]).wait()

    def drain_scatter(j, b):
        pltpu.make_async_copy(bufs[b], acc.at[rr[b]], ss[b]).wait()

    fire_idx(0, 0)
    fire_idx(1, 1)
    fire_idx(2, 2)
    wait_idx(0, 0)
    fire_gather(0, 0)
    wait_idx(1, 1)
    fire_gather(1, 1)

    def process(i, b):
        buf = bufs[b]
        wait_gather(i, b)
        done = 0
        while done < CHUNK:
            off = min(done, CHUNK - L)
            vv = vr[b][pl.ds(off, L)]
            for t in range(done - off, L):
                e = off + t
                vb = _bcast_lane(vv, t)
                for j in range(D // L):
                    sl = pl.ds(j * L, L)
                    buf[e, sl] = buf[e, sl] * vb
            done = off + L
        pltpu.async_copy(buf, acc.at[rr[b]], ss[b], add=True)

        b3 = (b + 3) % NBUF

        @pl.when(i + 3 < NCHUNK)
        def _idx_prefetch():
            @pl.when(i >= 1)
            def _drain():
                drain_scatter(i - 1, b3)
            fire_idx(i + 3, b3)

        b2 = (b + 2) % NBUF

        @pl.when(i + 2 < NCHUNK)
        def _gather_prefetch():
            wait_idx(i + 2, b2)
            fire_gather(i + 2, b2)

    def outer(k, carry):
        for b in range(NBUF):
            process(k * NBUF + b, b)
        return carry

    lax.fori_loop(0, NCHUNK // NBUF, outer, 0)
    process(NCHUNK - 1, (NCHUNK - 1) % NBUF)

    for b in range(NBUF):
        drain_scatter(NCHUNK - NBUF + b, (NCHUNK - NBUF + b) % NBUF)

    plsc.subcore_barrier()
    pltpu.sync_copy(acc.at[pl.ds(zbase, ROWS_PER_TILE)],
                    out_hbm.at[cid, pl.ds(zbase, ROWS_PER_TILE)])


_sc_edge = pl.kernel(
    _sc_edge_body,
    out_type=jax.ShapeDtypeStruct((NC, NP, D), jnp.float32),
    mesh=plsc.VectorSubcoreMesh(core_axis_name="c", subcore_axis_name="s",
                                num_cores=NC, num_subcores=NS),
    scratch_types=(
        [pltpu.VMEM((CHUNK,), jnp.int32) for _ in range(NBUF)]
        + [pltpu.VMEM((CHUNK,), jnp.int32) for _ in range(NBUF)]
        + [pltpu.VMEM((CHUNK,), jnp.float32) for _ in range(NBUF)]
        + [pltpu.VMEM((CHUNK, D), jnp.float32) for _ in range(NBUF)]
        + [pltpu.VMEM_SHARED((NP, D), jnp.float32)]
        + [pltpu.SemaphoreType.DMA for _ in range(3 * NBUF)]
    ),
)


def kernel(inputs, edge_row, edge_col, edge_val, W):
    h = _project(inputs, W)
    zeros = jnp.zeros((NP, D), jnp.float32)
    erow3 = edge_row.reshape(NW, NCHUNK, CHUNK)
    ecol3 = edge_col.reshape(NW, NCHUNK, CHUNK)
    eval3 = edge_val.reshape(NW, NCHUNK, CHUNK)
    partials = _sc_edge(h, erow3, ecol3, eval3, zeros)
    return _combine(partials)
